# baseline (speedup 1.0000x reference)
import functools

import jax
import jax.numpy as jnp
from jax import lax
from jax.experimental import pallas as pl
from jax.experimental.pallas import tpu as pltpu
from jax.experimental.pallas import tpu_sc as plsc

_D = 4096
_B = 8192
_NC, _NS = 2, 16
_N0 = 280
_N1 = 232
_R = 8
_NCH0 = _N0 // _R
_NCH1 = _N1 // _R
_NBUF = 3

_mesh = plsc.VectorSubcoreMesh(core_axis_name="c", subcore_axis_name="s")


@functools.partial(
    pl.kernel,
    mesh=_mesh,
    out_type=jax.ShapeDtypeStruct((_B, _D), jnp.float32),
    scratch_types=[
        pltpu.VMEM((_N0,), jnp.int32),
        pltpu.VMEM((_NBUF, _R, _D), jnp.float32),
    ]
    + [pltpu.SemaphoreType.DMA] * (2 * _NBUF),
)
def _gather_kernel(ids_hbm, table_hbm, out_hbm, idx_v, rows_v, *sems):
    gsems = sems[:_NBUF]
    ssems = sems[_NBUF:]
    cid = lax.axis_index("c")
    sid = lax.axis_index("s")
    base = sid * (_N0 + _N1) + (1 - cid) * _N0

    pltpu.sync_copy(ids_hbm.at[pl.ds(base, _N1)], idx_v.at[pl.ds(0, _N1)])

    def head(chunk, b):
        pltpu.async_copy(
            table_hbm.at[idx_v.at[pl.ds(chunk * _R, _R)]], rows_v.at[b], gsems[b]
        )

    def gather_wait(b):
        pltpu.make_async_copy(
            table_hbm.at[pl.ds(0, _R)], rows_v.at[b], gsems[b]
        ).wait()

    def start_scatter(chunk, b):
        pltpu.async_copy(
            rows_v.at[b], out_hbm.at[pl.ds(base + chunk * _R, _R)], ssems[b]
        )

    def scatter_wait(b):
        pltpu.make_async_copy(
            rows_v.at[b], out_hbm.at[pl.ds(base, _R)], ssems[b]
        ).wait()

    def core0_extra_ids():
        pltpu.sync_copy(
            ids_hbm.at[pl.ds(base + _N1, _N0 - _N1)],
            idx_v.at[pl.ds(_N1, _N0 - _N1)],
        )

    pl.when(cid == 1)(core0_extra_ids)

    for b in range(_NBUF):
        head(b, b)

    for chunk in range(_NCH0):
        b = chunk % _NBUF

        def step(chunk=chunk, b=b):
            gather_wait(b)
            start_scatter(chunk, b)
            nxt = chunk + _NBUF
            if nxt < _NCH0:
                lim = jnp.where(cid == 1, _NCH0, _NCH1)

                def refill(b=b, nxt=nxt):
                    scatter_wait(b)
                    head(nxt, b)

                pl.when(nxt < lim)(refill)

        if chunk < _NCH1:
            step()
        else:
            pl.when(cid == 1)(step)

    for b in range(_NBUF):
        scatter_wait(b)


def kernel(input_ids, word_embeddings):
    ids_flat = input_ids.reshape(-1).astype(jnp.int32)
    out = _gather_kernel(ids_flat, word_embeddings)
    return out.reshape(input_ids.shape + (word_embeddings.shape[1],))


# speedup vs baseline: 1.0125x; 1.0125x over previous
import functools

import jax
import jax.numpy as jnp
from jax import lax
from jax.experimental import pallas as pl
from jax.experimental.pallas import tpu as pltpu
from jax.experimental.pallas import tpu_sc as plsc

_D = 4096
_B = 8192
_NC, _NS = 2, 16
_NW = _NC * _NS
_B_PER_W = _B // _NW
_R = 8
_NCHUNK = _B_PER_W // _R
_NBUF = 3

_mesh = plsc.VectorSubcoreMesh(core_axis_name="c", subcore_axis_name="s")


@functools.partial(
    pl.kernel,
    mesh=_mesh,
    out_type=jax.ShapeDtypeStruct((_B, _D), jnp.float32),
    scratch_types=[
        pltpu.VMEM((_B_PER_W,), jnp.int32),
        pltpu.VMEM((_NBUF, _R, _D), jnp.float32),
    ]
    + [pltpu.SemaphoreType.DMA] * (2 * _NBUF),
)
def _gather_kernel(ids_hbm, table_hbm, out_hbm, idx_v, rows_v, *sems):
    gsems = sems[:_NBUF]
    ssems = sems[_NBUF:]
    wid = lax.axis_index("s") * _NC + lax.axis_index("c")
    base = wid * _B_PER_W
    pltpu.sync_copy(ids_hbm.at[pl.ds(base, _B_PER_W)], idx_v)

    def start_gather(chunk, b):
        pltpu.async_copy(
            table_hbm.at[idx_v.at[pl.ds(chunk * _R, _R)]], rows_v.at[b], gsems[b]
        )

    def gather_wait(b):
        pltpu.make_async_copy(
            table_hbm.at[pl.ds(0, _R)], rows_v.at[b], gsems[b]
        ).wait()

    def start_scatter(chunk, b):
        pltpu.async_copy(
            rows_v.at[b], out_hbm.at[pl.ds(base + chunk * _R, _R)], ssems[b]
        )

    def scatter_wait(b):
        pltpu.make_async_copy(
            rows_v.at[b], out_hbm.at[pl.ds(base, _R)], ssems[b]
        ).wait()

    for b in range(_NBUF):
        start_gather(b, b)

    for chunk in range(_NCHUNK):
        b = chunk % _NBUF
        gather_wait(b)
        start_scatter(chunk, b)
        nxt = chunk + _NBUF
        if nxt < _NCHUNK:
            scatter_wait(b)
            start_gather(nxt, b)

    for b in range(_NBUF):
        scatter_wait(b)


def kernel(input_ids, word_embeddings):
    ids_flat = input_ids.reshape(-1).astype(jnp.int32)
    out = _gather_kernel(ids_flat, word_embeddings)
    return out.reshape(input_ids.shape + (word_embeddings.shape[1],))
